# Initial kernel scaffold; baseline (speedup 1.0000x reference)
import functools

import jax
import jax.numpy as jnp
from jax import lax
from jax.experimental import pallas as pl
from jax.experimental.pallas import tpu as pltpu
from jax.experimental.pallas import tpu_sc as plsc

N = 4096
D = 128
HEADS = 4
DH = 32
E = 131072
K = 16
OUT = 128
TEMP = 0.5
NEG = 0.2
A = E + N * K + N

NC, NS, L = 2, 16, 16
NW = NC * NS

BR = 256
NB = N // BR
BR3 = 512


def _dot(a, b):
    return lax.dot_general(a, b, (((1,), (0,)), ((), ())),
                           preferred_element_type=jnp.float32)


def _dot_nt(a, b):
    return lax.dot_general(a, b, (((1,), (1,)), ((), ())),
                           preferred_element_type=jnp.float32)


def _t1_body(emb_blk, x_blk, emb_full, wq, wk, wn1, wa1,
             scores_out, wnew_out, topi_out, diag_out, ent_out,
             pdx_out, psx_out, pde_out, pse_out):
    i = pl.program_id(0)
    m = jnp.zeros((D, D), jnp.float32)
    for h in range(HEADS):
        m = m + _dot_nt(wq[h], wk[h])
    m = m / jnp.sqrt(jnp.float32(DH))
    p_blk = _dot(emb_blk[...], m)
    s = _dot_nt(p_blk, emb_full[...])
    scores_out[...] = s
    probs = jax.nn.sigmoid(s / TEMP)

    ent_part = -jnp.sum(probs * jnp.log(probs + 1e-10))

    @pl.when(i == 0)
    def _():
        ent_out[0, 0] = 0.0

    ent_out[0, 0] += ent_part

    col = lax.broadcasted_iota(jnp.int32, (BR, N), 1)
    row = lax.broadcasted_iota(jnp.int32, (BR, N), 0)
    diag_out[...] = jnp.sum(
        jnp.where(col == row + i * BR, probs, 0.0), axis=1, keepdims=True)

    work = probs
    tvs, tis = [], []
    for _k in range(K):
        mx = jnp.max(work, axis=1)
        sel = jnp.where(work == mx[:, None], col, N)
        ix = jnp.min(sel, axis=1)
        tvs.append(mx)
        tis.append(ix)
        work = jnp.where(col == ix[:, None], -1.0, work)
    topv = jnp.stack(tvs, axis=1)
    topi_out[...] = jnp.stack(tis, axis=1)
    wnew_out[...] = jnp.where(topv > 0.5, topv, 0.0)

    pdx_out[...] = _dot(x_blk[...], wn1[0:D, :])
    psx_out[...] = _dot(x_blk[...], wn1[D:2 * D, :])
    pde_out[...] = _dot(emb_blk[...], wa1[0:D, :])
    pse_out[...] = _dot(emb_blk[...], wa1[D:2 * D, :])


def _t1(emb, x, wq, wk, wn1, wa1):
    f32 = jnp.float32
    return pl.pallas_call(
        _t1_body,
        grid=(NB,),
        in_specs=[
            pl.BlockSpec((BR, D), lambda i: (i, 0)),
            pl.BlockSpec((BR, D), lambda i: (i, 0)),
            pl.BlockSpec((N, D), lambda i: (0, 0)),
            pl.BlockSpec((HEADS, D, DH), lambda i: (0, 0, 0)),
            pl.BlockSpec((HEADS, D, DH), lambda i: (0, 0, 0)),
            pl.BlockSpec((2 * D, D), lambda i: (0, 0)),
            pl.BlockSpec((2 * D, D), lambda i: (0, 0)),
        ],
        out_specs=[
            pl.BlockSpec((BR, N), lambda i: (i, 0)),
            pl.BlockSpec((BR, K), lambda i: (i, 0)),
            pl.BlockSpec((BR, K), lambda i: (i, 0)),
            pl.BlockSpec((BR, 1), lambda i: (i, 0)),
            pl.BlockSpec((1, 1), lambda i: (0, 0)),
            pl.BlockSpec((BR, D), lambda i: (i, 0)),
            pl.BlockSpec((BR, D), lambda i: (i, 0)),
            pl.BlockSpec((BR, D), lambda i: (i, 0)),
            pl.BlockSpec((BR, D), lambda i: (i, 0)),
        ],
        out_shape=[
            jax.ShapeDtypeStruct((N, N), f32),
            jax.ShapeDtypeStruct((N, K), f32),
            jax.ShapeDtypeStruct((N, K), jnp.int32),
            jax.ShapeDtypeStruct((N, 1), f32),
            jax.ShapeDtypeStruct((1, 1), f32),
            jax.ShapeDtypeStruct((N, D), f32),
            jax.ShapeDtypeStruct((N, D), f32),
            jax.ShapeDtypeStruct((N, D), f32),
            jax.ShapeDtypeStruct((N, D), f32),
        ],
    )(emb, x, emb, wq, wk, wn1, wa1)


_CH1 = 128
_PER1 = E // NW
_NCH1 = _PER1 // _CH1


def _sc_w_body(scores2d, srcs, dsts, out, srcv, dstv, idxv, rows, wv, sem):
    c = lax.axis_index("c")
    s = lax.axis_index("s")
    wid = s * NC + c
    base = wid * _PER1

    def chunk(ci, _):
        off = base + ci * _CH1
        pltpu.sync_copy(srcs.at[pl.ds(off, _CH1)], srcv)
        pltpu.sync_copy(dsts.at[pl.ds(off, _CH1)], dstv)

        def g16a(g, _):
            s16 = srcv[pl.ds(g * L, L)]
            d16 = dstv[pl.ds(g * L, L)]
            flat = s16 * N + d16
            idxv[pl.ds(g * L, L)] = lax.shift_right_logical(flat, 4)
            return 0

        lax.fori_loop(0, _CH1 // L, g16a, 0)
        pltpu.async_copy(scores2d.at[idxv], rows, sem).wait()

        def g16b(g, _):
            s16 = srcv[pl.ds(g * L, L)]
            d16 = dstv[pl.ds(g * L, L)]
            flat = s16 * N + d16
            lane = jnp.bitwise_and(flat, L - 1)
            rloc = lax.iota(jnp.int32, L) + g * L
            v = plsc.load_gather(rows, [rloc, lane])
            sig = 1.0 / (1.0 + jnp.exp(-(v / TEMP)))
            wv[pl.ds(g * L, L)] = jnp.where(s16 == d16, 0.0, sig)
            return 0

        lax.fori_loop(0, _CH1 // L, g16b, 0)
        pltpu.sync_copy(wv, out.at[pl.ds(off, _CH1)])
        return 0

    lax.fori_loop(0, _NCH1, chunk, 0)


def _sc_w(scores, src, dst):
    scores2d = scores.reshape(N * N // L, L)
    mesh = plsc.VectorSubcoreMesh(core_axis_name="c", subcore_axis_name="s",
                                  num_cores=NC, num_subcores=NS)
    f = pl.kernel(
        _sc_w_body,
        out_type=jax.ShapeDtypeStruct((E,), jnp.float32),
        mesh=mesh,
        scratch_types=[
            pltpu.VMEM((_CH1,), jnp.int32),
            pltpu.VMEM((_CH1,), jnp.int32),
            pltpu.VMEM((_CH1,), jnp.int32),
            pltpu.VMEM((_CH1, L), jnp.float32),
            pltpu.VMEM((_CH1,), jnp.float32),
            pltpu.SemaphoreType.DMA,
        ],
    )
    return f(scores2d, src, dst)


_CH2 = 128
_PER2 = A // NW
_NCH2 = _PER2 // _CH2
_ROWS = N // NS


def _sc_msg_body(pdx, psx, pde, pse, asrc, adst, aw,
                 accx_out, acce_out, cnt_out,
                 srcv, dstv, wv, bpdx, bpsx, bpde, bpse, mx, me, onesb, zb,
                 shx, she, shc, sem0, sem1, sem2, sem3):
    c = lax.axis_index("c")
    s = lax.axis_index("s")
    wid = s * NC + c

    def zrow(r, _):
        for j in range(D // L):
            mx[r, pl.ds(j * L, L)] = jnp.zeros((L,), jnp.float32)
        onesb[r, :] = jnp.ones((L,), jnp.float32)
        return 0

    lax.fori_loop(0, _CH2, zrow, 0)

    def zrow16(r, _):
        zb[r, :] = jnp.zeros((L,), jnp.float32)
        return 0

    lax.fori_loop(0, _ROWS, zrow16, 0)

    pltpu.sync_copy(mx, shx.at[pl.ds(s * _ROWS, _CH2)])
    pltpu.sync_copy(mx, shx.at[pl.ds(s * _ROWS + _CH2, _CH2)])
    pltpu.sync_copy(mx, she.at[pl.ds(s * _ROWS, _CH2)])
    pltpu.sync_copy(mx, she.at[pl.ds(s * _ROWS + _CH2, _CH2)])
    pltpu.sync_copy(zb, shc.at[pl.ds(s * _ROWS, _ROWS)])
    plsc.subcore_barrier()

    base = wid * _PER2

    def chunk(ci, _):
        off = base + ci * _CH2
        pltpu.sync_copy(asrc.at[pl.ds(off, _CH2)], srcv)
        pltpu.sync_copy(adst.at[pl.ds(off, _CH2)], dstv)
        pltpu.sync_copy(aw.at[pl.ds(off, _CH2)], wv)
        cp0 = pltpu.async_copy(pdx.at[dstv], bpdx, sem0)
        cp1 = pltpu.async_copy(psx.at[srcv], bpsx, sem1)
        cp2 = pltpu.async_copy(pde.at[dstv], bpde, sem2)
        cp3 = pltpu.async_copy(pse.at[srcv], bpse, sem3)
        cp0.wait()
        cp1.wait()
        cp2.wait()
        cp3.wait()

        def edge(e, _):
            w = wv[e]
            for j in range(D // L):
                a = bpdx[e, pl.ds(j * L, L)] + bpsx[e, pl.ds(j * L, L)]
                a = jnp.maximum(a, NEG * a)
                mx[e, pl.ds(j * L, L)] = a * w
                b = bpde[e, pl.ds(j * L, L)] + bpse[e, pl.ds(j * L, L)]
                b = jnp.maximum(b, NEG * b)
                me[e, pl.ds(j * L, L)] = b * w
            return 0

        lax.fori_loop(0, _CH2, edge, 0)
        pltpu.sync_copy(mx, shx.at[dstv], add=True)
        pltpu.sync_copy(me, she.at[dstv], add=True)
        pltpu.sync_copy(onesb, shc.at[dstv], add=True)
        return 0

    lax.fori_loop(0, _NCH2, chunk, 0)
    plsc.subcore_barrier()

    pltpu.sync_copy(shx.at[pl.ds(s * _ROWS, _ROWS)],
                    accx_out.at[c, pl.ds(s * _ROWS, _ROWS)])
    pltpu.sync_copy(she.at[pl.ds(s * _ROWS, _ROWS)],
                    acce_out.at[c, pl.ds(s * _ROWS, _ROWS)])
    pltpu.sync_copy(shc.at[pl.ds(s * _ROWS, _ROWS)],
                    cnt_out.at[c, pl.ds(s * _ROWS, _ROWS)])


def _sc_msg(pdx, psx, pde, pse, all_src, all_dst, all_w):
    f32 = jnp.float32
    mesh = plsc.VectorSubcoreMesh(core_axis_name="c", subcore_axis_name="s",
                                  num_cores=NC, num_subcores=NS)
    f = pl.kernel(
        _sc_msg_body,
        out_type=[
            jax.ShapeDtypeStruct((NC, N, D), f32),
            jax.ShapeDtypeStruct((NC, N, D), f32),
            jax.ShapeDtypeStruct((NC, N, L), f32),
        ],
        mesh=mesh,
        scratch_types=[
            pltpu.VMEM((_CH2,), jnp.int32),
            pltpu.VMEM((_CH2,), jnp.int32),
            pltpu.VMEM((_CH2,), f32),
            pltpu.VMEM((_CH2, D), f32),
            pltpu.VMEM((_CH2, D), f32),
            pltpu.VMEM((_CH2, D), f32),
            pltpu.VMEM((_CH2, D), f32),
            pltpu.VMEM((_CH2, D), f32),
            pltpu.VMEM((_CH2, D), f32),
            pltpu.VMEM((_CH2, L), f32),
            pltpu.VMEM((_ROWS, L), f32),
            pltpu.VMEM_SHARED((N, D), f32),
            pltpu.VMEM_SHARED((N, D), f32),
            pltpu.VMEM_SHARED((N, L), f32),
            pltpu.SemaphoreType.DMA,
            pltpu.SemaphoreType.DMA,
            pltpu.SemaphoreType.DMA,
            pltpu.SemaphoreType.DMA,
        ],
    )
    return f(pdx, psx, pde, pse, all_src, all_dst, all_w)


def _t3_body(accx, acce, cnt, x_blk, emb_blk, wn2, wa2, wt1, wt2, wm1, wm2,
             oval, oatt):
    cx = accx[0] + accx[1]
    ce = acce[0] + acce[1]
    cdeg = cnt[0, :, 0:1] + cnt[1, :, 0:1]
    meanx = _dot(cx, wn2[...]) / cdeg
    h = _dot(meanx, wt1[0:D, :]) + _dot(x_blk[...], wt1[D:2 * D, :])
    h = jnp.where(h >= 0, h, NEG * h)
    oval[...] = _dot(h, wt2[...])
    meane = _dot(ce, wa2[...]) / cdeg
    g = _dot(meane, wm1[0:D, :]) + _dot(emb_blk[...], wm1[D:2 * D, :])
    g = jnp.where(g >= 0, g, NEG * g)
    oatt[...] = _dot(g, wm2[...])


def _t3(accx, acce, cnt, x, emb, wn2, wa2, wt1, wt2, wm1, wm2):
    f32 = jnp.float32
    nb = N // BR3
    return pl.pallas_call(
        _t3_body,
        grid=(nb,),
        in_specs=[
            pl.BlockSpec((NC, BR3, D), lambda i: (0, i, 0)),
            pl.BlockSpec((NC, BR3, D), lambda i: (0, i, 0)),
            pl.BlockSpec((NC, BR3, L), lambda i: (0, i, 0)),
            pl.BlockSpec((BR3, D), lambda i: (i, 0)),
            pl.BlockSpec((BR3, D), lambda i: (i, 0)),
            pl.BlockSpec((D, D), lambda i: (0, 0)),
            pl.BlockSpec((D, D), lambda i: (0, 0)),
            pl.BlockSpec((2 * D, D), lambda i: (0, 0)),
            pl.BlockSpec((D, OUT), lambda i: (0, 0)),
            pl.BlockSpec((2 * D, D), lambda i: (0, 0)),
            pl.BlockSpec((D, OUT), lambda i: (0, 0)),
        ],
        out_specs=[
            pl.BlockSpec((BR3, OUT), lambda i: (i, 0)),
            pl.BlockSpec((BR3, OUT), lambda i: (i, 0)),
        ],
        out_shape=[
            jax.ShapeDtypeStruct((N, OUT), f32),
            jax.ShapeDtypeStruct((N, OUT), f32),
        ],
    )(accx, acce, cnt, x, emb, wn2, wa2, wt1, wt2, wm1, wm2)


def kernel(x, decoupled_emb, edge_index, mask, attention_init,
           Wq, Wk, Wn1, Wn2, Wa1, Wa2, Wt1, Wt2, Wm1, Wm2):
    emb = decoupled_emb
    (scores, wnew, topi, diag, ent,
     pdx, psx, pde, pse) = _t1(emb, x, Wq, Wk, Wn1, Wa1)

    src = edge_index[0]
    dst = edge_index[1]
    w_orig = _sc_w(scores, src, dst)

    loop = jnp.arange(N, dtype=jnp.int32)
    all_src = jnp.concatenate([src, jnp.repeat(loop, K), loop])
    all_dst = jnp.concatenate([dst, topi.reshape(-1), loop])
    all_w = jnp.concatenate([w_orig, wnew.reshape(-1), diag.reshape(-1)])

    accx, acce, cnt = _sc_msg(pdx, psx, pde, pse, all_src, all_dst, all_w)
    out_val, out_att = _t3(accx, acce, cnt, x, emb,
                           Wn2, Wa2, Wt1, Wt2, Wm1, Wm2)

    updated_edge_index = jnp.stack([all_src, all_dst])
    edge_penalty = ent[0, 0]
    return out_val, updated_edge_index, edge_penalty, scores, out_att, all_w


# speedup vs baseline: 7.3874x; 7.3874x over previous
import jax
import jax.numpy as jnp
from jax import lax
from jax.experimental import pallas as pl
from jax.experimental.pallas import tpu as pltpu
from jax.experimental.pallas import tpu_sc as plsc

N = 4096
D = 128
HEADS = 4
DH = 32
E = 131072
K = 16
OUT = 128
TEMP = 0.5
NEG = 0.2
A = E + N * K + N

NC, NS, L = 2, 16, 16
NW = NC * NS

BR = 256
NB = N // BR
BR3 = 512


def _dot(a, b):
    return lax.dot_general(a, b, (((1,), (0,)), ((), ())),
                           preferred_element_type=jnp.float32)


def _dot_nt(a, b):
    return lax.dot_general(a, b, (((1,), (1,)), ((), ())),
                           preferred_element_type=jnp.float32)


def _t1_body(emb_blk, x_blk, emb_full, wq, wk, wn1, wa1,
             scores_out, wnew_out, topi_out, diag_out, ent_out,
             pdx_out, psx_out, pde_out, pse_out):
    i = pl.program_id(0)
    m = jnp.zeros((D, D), jnp.float32)
    for h in range(HEADS):
        m = m + _dot_nt(wq[h], wk[h])
    m = m / jnp.sqrt(jnp.float32(DH))
    p_blk = _dot(emb_blk[...], m)
    s = _dot_nt(p_blk, emb_full[...])
    scores_out[...] = s
    probs = jax.nn.sigmoid(s / TEMP)

    ent_part = jnp.full((1, 1), -jnp.sum(probs * jnp.log(probs + 1e-10)),
                        jnp.float32)

    @pl.when(i == 0)
    def _():
        ent_out[...] = ent_part

    @pl.when(i != 0)
    def _():
        ent_out[...] += ent_part

    col = lax.broadcasted_iota(jnp.int32, (BR, N), 1)
    row = lax.broadcasted_iota(jnp.int32, (BR, N), 0)
    diag_out[...] = jnp.sum(
        jnp.where(col == row + i * BR, probs, 0.0), axis=1, keepdims=True)

    work = probs
    tvs, tis = [], []
    for _k in range(K):
        mx = jnp.max(work, axis=1)
        sel = jnp.where(work == mx[:, None], col, N)
        ix = jnp.min(sel, axis=1)
        tvs.append(mx)
        tis.append(ix)
        work = jnp.where(col == ix[:, None], -1.0, work)
    topv = jnp.stack(tvs, axis=1)
    topi_out[...] = jnp.stack(tis, axis=1)
    wnew_out[...] = jnp.where(topv > 0.5, topv, 0.0)

    pdx_out[...] = _dot(x_blk[...], wn1[0:D, :])
    psx_out[...] = _dot(x_blk[...], wn1[D:2 * D, :])
    pde_out[...] = _dot(emb_blk[...], wa1[0:D, :])
    pse_out[...] = _dot(emb_blk[...], wa1[D:2 * D, :])


def _t1(emb, x, wq, wk, wn1, wa1):
    f32 = jnp.float32
    return pl.pallas_call(
        _t1_body,
        grid=(NB,),
        in_specs=[
            pl.BlockSpec((BR, D), lambda i: (i, 0)),
            pl.BlockSpec((BR, D), lambda i: (i, 0)),
            pl.BlockSpec((N, D), lambda i: (0, 0)),
            pl.BlockSpec((HEADS, D, DH), lambda i: (0, 0, 0)),
            pl.BlockSpec((HEADS, D, DH), lambda i: (0, 0, 0)),
            pl.BlockSpec((2 * D, D), lambda i: (0, 0)),
            pl.BlockSpec((2 * D, D), lambda i: (0, 0)),
        ],
        out_specs=[
            pl.BlockSpec((BR, N), lambda i: (i, 0)),
            pl.BlockSpec((BR, K), lambda i: (i, 0)),
            pl.BlockSpec((BR, K), lambda i: (i, 0)),
            pl.BlockSpec((BR, 1), lambda i: (i, 0)),
            pl.BlockSpec((1, 1), lambda i: (0, 0)),
            pl.BlockSpec((BR, D), lambda i: (i, 0)),
            pl.BlockSpec((BR, D), lambda i: (i, 0)),
            pl.BlockSpec((BR, D), lambda i: (i, 0)),
            pl.BlockSpec((BR, D), lambda i: (i, 0)),
        ],
        out_shape=[
            jax.ShapeDtypeStruct((N, N), f32),
            jax.ShapeDtypeStruct((N, K), f32),
            jax.ShapeDtypeStruct((N, K), jnp.int32),
            jax.ShapeDtypeStruct((N, 1), f32),
            jax.ShapeDtypeStruct((1, 1), f32),
            jax.ShapeDtypeStruct((N, D), f32),
            jax.ShapeDtypeStruct((N, D), f32),
            jax.ShapeDtypeStruct((N, D), f32),
            jax.ShapeDtypeStruct((N, D), f32),
        ],
    )(emb, x, emb, wq, wk, wn1, wa1)


_C1 = 128
_PER1 = E // NW
_NCH1 = _PER1 // _C1


def _sc_w_body(scores1d, srcs, dsts, wout, srcv, dstv, idxv, rawv, wv, sem):
    c = lax.axis_index("c")
    s = lax.axis_index("s")
    wid = s * NC + c
    base = wid * _PER1

    def chunk(ci, _):
        off = base + ci * _C1
        pltpu.sync_copy(srcs.at[pl.ds(off, _C1)], srcv)
        pltpu.sync_copy(dsts.at[pl.ds(off, _C1)], dstv)
        for g in range(_C1 // L):
            sl = pl.ds(g * L, L)
            idxv[sl] = srcv[sl] * N + dstv[sl]
        pltpu.async_copy(scores1d.at[idxv], rawv, sem).wait()
        for g in range(_C1 // L):
            sl = pl.ds(g * L, L)
            sig = 1.0 / (1.0 + jnp.exp(-(rawv[sl] / TEMP)))
            wv[sl] = jnp.where(srcv[sl] == dstv[sl], 0.0, sig)
        pltpu.sync_copy(wv, wout.at[pl.ds(off, _C1)])
        return 0

    lax.fori_loop(0, _NCH1, chunk, 0)


def _sc_w(scores, src, dst):
    scores1d = scores.reshape(N * N)
    mesh = plsc.VectorSubcoreMesh(core_axis_name="c", subcore_axis_name="s",
                                  num_cores=NC, num_subcores=NS)
    f = pl.kernel(
        _sc_w_body,
        out_type=jax.ShapeDtypeStruct((E,), jnp.float32),
        mesh=mesh,
        scratch_types=[
            pltpu.VMEM((_C1,), jnp.int32),
            pltpu.VMEM((_C1,), jnp.int32),
            pltpu.VMEM((_C1,), jnp.int32),
            pltpu.VMEM((_C1,), jnp.float32),
            pltpu.VMEM((_C1,), jnp.float32),
            pltpu.SemaphoreType.DMA,
        ],
    )
    return f(scores1d, src, dst)


_C2 = 32
_PER2 = A // NW
_NCH2 = _PER2 // _C2
_STR = N // NS


def _sc_msg_body(pdx, psx, pde, pse, asrc, adst, aw, zd, zc,
                 accx_out, acce_out, cnt_out,
                 srcv, dstv, wv, bxd, bxs, bed, bes, onesb,
                 shx, she, shc, sem0, sem1, sem2, sem3):
    c = lax.axis_index("c")
    s = lax.axis_index("s")
    wid = s * NC + c
    base = wid * _PER2
    row0 = s * _STR

    for r in range(_C2):
        onesb[r, :] = jnp.ones((L,), jnp.float32)

    pltpu.sync_copy(zd.at[pl.ds(row0, _STR)], shx.at[pl.ds(row0, _STR)])
    pltpu.sync_copy(zd.at[pl.ds(row0, _STR)], she.at[pl.ds(row0, _STR)])
    pltpu.sync_copy(zc.at[pl.ds(row0, _STR)], shc.at[pl.ds(row0, _STR)])
    plsc.subcore_barrier()

    def chunk(ci, _):
        off = base + ci * _C2
        pltpu.sync_copy(asrc.at[pl.ds(off, _C2)], srcv)
        pltpu.sync_copy(adst.at[pl.ds(off, _C2)], dstv)
        pltpu.sync_copy(aw.at[pl.ds(off, _C2)], wv)
        cpa = pltpu.async_copy(pdx.at[dstv], bxd, sem0)
        cpb = pltpu.async_copy(psx.at[srcv], bxs, sem1)
        cpc = pltpu.async_copy(pde.at[dstv], bed, sem2)
        cpd = pltpu.async_copy(pse.at[srcv], bes, sem3)
        cpa.wait()
        cpb.wait()
        cpc.wait()
        cpd.wait()
        for g in range(_C2 // L):
            wv16 = wv[pl.ds(g * L, L)]
            for el in range(L):
                e = g * L + el
                w = wv16[el]
                for j in range(D // L):
                    sl = pl.ds(j * L, L)
                    ax = bxd[e, sl] + bxs[e, sl]
                    bxd[e, sl] = jnp.maximum(ax, NEG * ax) * w
                    ae = bed[e, sl] + bes[e, sl]
                    bed[e, sl] = jnp.maximum(ae, NEG * ae) * w
        pltpu.sync_copy(bxd, shx.at[dstv], add=True)
        pltpu.sync_copy(bed, she.at[dstv], add=True)
        pltpu.sync_copy(onesb, shc.at[dstv], add=True)
        return 0

    lax.fori_loop(0, _NCH2, chunk, 0)
    plsc.subcore_barrier()

    pltpu.sync_copy(shx.at[pl.ds(row0, _STR)],
                    accx_out.at[c, pl.ds(row0, _STR)])
    pltpu.sync_copy(she.at[pl.ds(row0, _STR)],
                    acce_out.at[c, pl.ds(row0, _STR)])
    pltpu.sync_copy(shc.at[pl.ds(row0, _STR)],
                    cnt_out.at[c, pl.ds(row0, _STR)])


def _sc_msg(pdx, psx, pde, pse, all_src, all_dst, all_w):
    f32 = jnp.float32
    zd = jnp.zeros((N, D), f32)
    zc = jnp.zeros((N, L), f32)
    mesh = plsc.VectorSubcoreMesh(core_axis_name="c", subcore_axis_name="s",
                                  num_cores=NC, num_subcores=NS)
    f = pl.kernel(
        _sc_msg_body,
        out_type=[
            jax.ShapeDtypeStruct((NC, N, D), f32),
            jax.ShapeDtypeStruct((NC, N, D), f32),
            jax.ShapeDtypeStruct((NC, N, L), f32),
        ],
        mesh=mesh,
        scratch_types=[
            pltpu.VMEM((_C2,), jnp.int32),
            pltpu.VMEM((_C2,), jnp.int32),
            pltpu.VMEM((_C2,), f32),
            pltpu.VMEM((_C2, D), f32),
            pltpu.VMEM((_C2, D), f32),
            pltpu.VMEM((_C2, D), f32),
            pltpu.VMEM((_C2, D), f32),
            pltpu.VMEM((_C2, L), f32),
            pltpu.VMEM_SHARED((N, D), f32),
            pltpu.VMEM_SHARED((N, D), f32),
            pltpu.VMEM_SHARED((N, L), f32),
            pltpu.SemaphoreType.DMA,
            pltpu.SemaphoreType.DMA,
            pltpu.SemaphoreType.DMA,
            pltpu.SemaphoreType.DMA,
        ],
    )
    return f(pdx, psx, pde, pse, all_src, all_dst, all_w, zd, zc)


def _t3_body(accx, acce, cnt, x_blk, emb_blk, wn2, wa2, wt1, wt2, wm1, wm2,
             oval, oatt):
    cx = accx[0] + accx[1]
    ce = acce[0] + acce[1]
    cdeg = cnt[0, :, 0:1] + cnt[1, :, 0:1]
    meanx = _dot(cx, wn2[...]) / cdeg
    h = _dot(meanx, wt1[0:D, :]) + _dot(x_blk[...], wt1[D:2 * D, :])
    h = jnp.where(h >= 0, h, NEG * h)
    oval[...] = _dot(h, wt2[...])
    meane = _dot(ce, wa2[...]) / cdeg
    g = _dot(meane, wm1[0:D, :]) + _dot(emb_blk[...], wm1[D:2 * D, :])
    g = jnp.where(g >= 0, g, NEG * g)
    oatt[...] = _dot(g, wm2[...])


def _t3(accx, acce, cnt, x, emb, wn2, wa2, wt1, wt2, wm1, wm2):
    f32 = jnp.float32
    nb = N // BR3
    return pl.pallas_call(
        _t3_body,
        grid=(nb,),
        in_specs=[
            pl.BlockSpec((NC, BR3, D), lambda i: (0, i, 0)),
            pl.BlockSpec((NC, BR3, D), lambda i: (0, i, 0)),
            pl.BlockSpec((NC, BR3, L), lambda i: (0, i, 0)),
            pl.BlockSpec((BR3, D), lambda i: (i, 0)),
            pl.BlockSpec((BR3, D), lambda i: (i, 0)),
            pl.BlockSpec((D, D), lambda i: (0, 0)),
            pl.BlockSpec((D, D), lambda i: (0, 0)),
            pl.BlockSpec((2 * D, D), lambda i: (0, 0)),
            pl.BlockSpec((D, OUT), lambda i: (0, 0)),
            pl.BlockSpec((2 * D, D), lambda i: (0, 0)),
            pl.BlockSpec((D, OUT), lambda i: (0, 0)),
        ],
        out_specs=[
            pl.BlockSpec((BR3, OUT), lambda i: (i, 0)),
            pl.BlockSpec((BR3, OUT), lambda i: (i, 0)),
        ],
        out_shape=[
            jax.ShapeDtypeStruct((N, OUT), f32),
            jax.ShapeDtypeStruct((N, OUT), f32),
        ],
    )(accx, acce, cnt, x, emb, wn2, wa2, wt1, wt2, wm1, wm2)


def kernel(x, decoupled_emb, edge_index, mask, attention_init,
           Wq, Wk, Wn1, Wn2, Wa1, Wa2, Wt1, Wt2, Wm1, Wm2):
    emb = decoupled_emb
    (scores, wnew, topi, diag, ent,
     pdx, psx, pde, pse) = _t1(emb, x, Wq, Wk, Wn1, Wa1)

    src = edge_index[0]
    dst = edge_index[1]
    w_orig = _sc_w(scores, src, dst)

    loop = jnp.arange(N, dtype=jnp.int32)
    all_src = jnp.concatenate([src, jnp.repeat(loop, K), loop])
    all_dst = jnp.concatenate([dst, topi.reshape(-1), loop])
    all_w = jnp.concatenate([w_orig, wnew.reshape(-1), diag.reshape(-1)])

    accx, acce, cnt = _sc_msg(pdx, psx, pde, pse, all_src, all_dst, all_w)
    out_val, out_att = _t3(accx, acce, cnt, x, emb,
                           Wn2, Wa2, Wt1, Wt2, Wm1, Wm2)

    updated_edge_index = jnp.stack([all_src, all_dst])
    edge_penalty = ent[0, 0]
    return out_val, updated_edge_index, edge_penalty, scores, out_att, all_w


# speedup vs baseline: 8.0959x; 1.0959x over previous
import jax
import jax.numpy as jnp
from jax import lax
from jax.experimental import pallas as pl
from jax.experimental.pallas import tpu as pltpu
from jax.experimental.pallas import tpu_sc as plsc

N = 4096
D = 128
HEADS = 4
DH = 32
E = 131072
K = 16
OUT = 128
TEMP = 0.5
NEG = 0.2
A = E + N * K + N

NC, NS, L = 2, 16, 16
NW = NC * NS

BR = 256
NB = N // BR
BR3 = 512


def _dot(a, b):
    return lax.dot_general(a, b, (((1,), (0,)), ((), ())),
                           preferred_element_type=jnp.float32)


def _dot_nt(a, b):
    return lax.dot_general(a, b, (((1,), (1,)), ((), ())),
                           preferred_element_type=jnp.float32)


def _t1_body(emb_blk, x_blk, emb_full, wq, wk, wn1, wa1,
             scores_out, wnew_out, topi_out, diag_out, ent_out,
             pdx_out, psx_out):
    i = pl.program_id(0)
    m = jnp.zeros((D, D), jnp.float32)
    for h in range(HEADS):
        m = m + _dot_nt(wq[h], wk[h])
    m = m / jnp.sqrt(jnp.float32(DH))
    p_blk = _dot(emb_blk[...], m)
    s = _dot_nt(p_blk, emb_full[...])
    scores_out[...] = s
    probs = jax.nn.sigmoid(s / TEMP)

    ent_part = jnp.full((1, 1), -jnp.sum(probs * jnp.log(probs + 1e-10)),
                        jnp.float32)

    @pl.when(i == 0)
    def _():
        ent_out[...] = ent_part

    @pl.when(i != 0)
    def _():
        ent_out[...] += ent_part

    col = lax.broadcasted_iota(jnp.int32, (BR, N), 1)
    row = lax.broadcasted_iota(jnp.int32, (BR, N), 0)
    diag_out[...] = jnp.sum(
        jnp.where(col == row + i * BR, probs, 0.0), axis=1, keepdims=True)

    work = probs
    tvs, tis = [], []
    for _k in range(K):
        mx = jnp.max(work, axis=1)
        sel = jnp.where(work == mx[:, None], col, N)
        ix = jnp.min(sel, axis=1)
        tvs.append(mx)
        tis.append(ix)
        work = jnp.where(col == ix[:, None], -1.0, work)
    topv = jnp.stack(tvs, axis=1)
    topi_out[...] = jnp.stack(tis, axis=1)
    wnew_out[...] = jnp.where(topv > 0.5, topv, 0.0)

    pdx_out[:, 0:D] = _dot(x_blk[...], wn1[0:D, :])
    pdx_out[:, D:2 * D] = _dot(emb_blk[...], wa1[0:D, :])
    psx_out[:, 0:D] = _dot(x_blk[...], wn1[D:2 * D, :])
    psx_out[:, D:2 * D] = _dot(emb_blk[...], wa1[D:2 * D, :])


def _t1(emb, x, wq, wk, wn1, wa1):
    f32 = jnp.float32
    return pl.pallas_call(
        _t1_body,
        grid=(NB,),
        in_specs=[
            pl.BlockSpec((BR, D), lambda i: (i, 0)),
            pl.BlockSpec((BR, D), lambda i: (i, 0)),
            pl.BlockSpec((N, D), lambda i: (0, 0)),
            pl.BlockSpec((HEADS, D, DH), lambda i: (0, 0, 0)),
            pl.BlockSpec((HEADS, D, DH), lambda i: (0, 0, 0)),
            pl.BlockSpec((2 * D, D), lambda i: (0, 0)),
            pl.BlockSpec((2 * D, D), lambda i: (0, 0)),
        ],
        out_specs=[
            pl.BlockSpec((BR, N), lambda i: (i, 0)),
            pl.BlockSpec((BR, K), lambda i: (i, 0)),
            pl.BlockSpec((BR, K), lambda i: (i, 0)),
            pl.BlockSpec((BR, 1), lambda i: (i, 0)),
            pl.BlockSpec((1, 1), lambda i: (0, 0)),
            pl.BlockSpec((BR, 2 * D), lambda i: (i, 0)),
            pl.BlockSpec((BR, 2 * D), lambda i: (i, 0)),
        ],
        out_shape=[
            jax.ShapeDtypeStruct((N, N), f32),
            jax.ShapeDtypeStruct((N, K), f32),
            jax.ShapeDtypeStruct((N, K), jnp.int32),
            jax.ShapeDtypeStruct((N, 1), f32),
            jax.ShapeDtypeStruct((1, 1), f32),
            jax.ShapeDtypeStruct((N, 2 * D), f32),
            jax.ShapeDtypeStruct((N, 2 * D), f32),
        ],
    )(emb, x, emb, wq, wk, wn1, wa1)


_C1 = 128
_PER1 = E // NW
_NCH1 = _PER1 // _C1


def _sc_w_body(scores1d, srcs, dsts, wout, srcv, dstv, idxv, rawv, wv, sem):
    c = lax.axis_index("c")
    s = lax.axis_index("s")
    wid = s * NC + c
    base = wid * _PER1

    def chunk(ci, _):
        off = base + ci * _C1
        pltpu.sync_copy(srcs.at[pl.ds(off, _C1)], srcv)
        pltpu.sync_copy(dsts.at[pl.ds(off, _C1)], dstv)
        for g in range(_C1 // L):
            sl = pl.ds(g * L, L)
            idxv[sl] = srcv[sl] * N + dstv[sl]
        pltpu.async_copy(scores1d.at[idxv], rawv, sem).wait()
        for g in range(_C1 // L):
            sl = pl.ds(g * L, L)
            sig = 1.0 / (1.0 + jnp.exp(-(rawv[sl] / TEMP)))
            wv[sl] = jnp.where(srcv[sl] == dstv[sl], 0.0, sig)
        pltpu.sync_copy(wv, wout.at[pl.ds(off, _C1)])
        return 0

    lax.fori_loop(0, _NCH1, chunk, 0)


def _sc_w(scores, src, dst):
    scores1d = scores.reshape(N * N)
    mesh = plsc.VectorSubcoreMesh(core_axis_name="c", subcore_axis_name="s",
                                  num_cores=NC, num_subcores=NS)
    f = pl.kernel(
        _sc_w_body,
        out_type=jax.ShapeDtypeStruct((E,), jnp.float32),
        mesh=mesh,
        scratch_types=[
            pltpu.VMEM((_C1,), jnp.int32),
            pltpu.VMEM((_C1,), jnp.int32),
            pltpu.VMEM((_C1,), jnp.int32),
            pltpu.VMEM((_C1,), jnp.float32),
            pltpu.VMEM((_C1,), jnp.float32),
            pltpu.SemaphoreType.DMA,
        ],
    )
    return f(scores1d, src, dst)


_C2 = 16
_PER2 = A // NW
_NCH2 = _PER2 // _C2
_W2 = 2 * D


def _sc2_compute(wb, bd, bs, box, boe):
    for g in range(_C2 // L):
        wv16 = wb[pl.ds(g * L, L)]
        for el in range(L):
            e = g * L + el
            w = wv16[el]
            for j in range(D // L):
                sl = pl.ds(j * L, L)
                sl2 = pl.ds(D + j * L, L)
                a = bd[e, sl] + bs[e, sl]
                box[e, sl] = jnp.maximum(a, NEG * a) * w
                a2 = bd[e, sl2] + bs[e, sl2]
                boe[e, sl] = jnp.maximum(a2, NEG * a2) * w


def _sc_msg_body(pdx, psx, idxpk, wpk, zrow, zcnt,
                 accx_out, acce_out, cnt_out,
                 ib0, ib1, wb0, wb1, bd0, bs0, bd1, bs1,
                 box0, boe0, box1, boe1, onesb,
                 shx, she, shc, sg0, sg1, sg2, sg3,
                 ss0, ss1, ss2, ss3, ss4, ss5):
    c = lax.axis_index("c")
    s = lax.axis_index("s")
    wid = s * NC + c
    cbase = wid * _NCH2
    row0 = s * (N // NS)

    ones = jnp.ones((L,), jnp.float32)
    for e in range(_C2):
        onesb[e, :] = ones

    pltpu.sync_copy(zrow.at[pl.ds(row0, N // NS)],
                    shx.at[pl.ds(row0, N // NS)])
    pltpu.sync_copy(zrow.at[pl.ds(row0, N // NS)],
                    she.at[pl.ds(row0, N // NS)])
    pltpu.sync_copy(zcnt.at[pl.ds(row0, N // NS)],
                    shc.at[pl.ds(row0, N // NS)])
    plsc.subcore_barrier()

    def pair(pi, _):
        c0 = cbase + pi * 2
        pltpu.sync_copy(idxpk.at[c0], ib0)
        pltpu.sync_copy(wpk.at[c0], wb0)
        g0a = pltpu.async_copy(pdx.at[ib0.at[1]], bd0, sg0)
        g0b = pltpu.async_copy(psx.at[ib0.at[0]], bs0, sg1)
        pltpu.sync_copy(idxpk.at[c0 + 1], ib1)
        pltpu.sync_copy(wpk.at[c0 + 1], wb1)
        g1a = pltpu.async_copy(pdx.at[ib1.at[1]], bd1, sg2)
        g1b = pltpu.async_copy(psx.at[ib1.at[0]], bs1, sg3)
        g0a.wait()
        g0b.wait()
        _sc2_compute(wb0, bd0, bs0, box0, boe0)
        sc0 = pltpu.async_copy(box0, shx.at[ib0.at[1]], ss0, add=True)
        se0 = pltpu.async_copy(boe0, she.at[ib0.at[1]], ss1, add=True)
        sn0 = pltpu.async_copy(onesb, shc.at[ib0.at[1]], ss2, add=True)
        g1a.wait()
        g1b.wait()
        _sc2_compute(wb1, bd1, bs1, box1, boe1)
        sc1 = pltpu.async_copy(box1, shx.at[ib1.at[1]], ss3, add=True)
        se1 = pltpu.async_copy(boe1, she.at[ib1.at[1]], ss4, add=True)
        sn1 = pltpu.async_copy(onesb, shc.at[ib1.at[1]], ss5, add=True)
        sc0.wait()
        se0.wait()
        sn0.wait()
        sc1.wait()
        se1.wait()
        sn1.wait()
        return 0

    lax.fori_loop(0, _NCH2 // 2, pair, 0)
    plsc.subcore_barrier()

    pltpu.sync_copy(shx.at[pl.ds(row0, N // NS)],
                    accx_out.at[c, pl.ds(row0, N // NS)])
    pltpu.sync_copy(she.at[pl.ds(row0, N // NS)],
                    acce_out.at[c, pl.ds(row0, N // NS)])
    pltpu.sync_copy(shc.at[pl.ds(row0, N // NS)],
                    cnt_out.at[c, pl.ds(row0, N // NS)])


def _sc_msg(pdx, psx, all_src, all_dst, all_w):
    f32 = jnp.float32
    i32 = jnp.int32
    nch = A // _C2
    idxpk = jnp.stack([all_src, all_dst])
    idxpk = idxpk.reshape(2, nch, _C2).transpose(1, 0, 2)
    wpk = all_w.reshape(nch, _C2)
    zrow = jnp.zeros((N, D), f32)
    zcnt = jnp.zeros((N, L), f32)
    mesh = plsc.VectorSubcoreMesh(core_axis_name="c", subcore_axis_name="s",
                                  num_cores=NC, num_subcores=NS)
    f = pl.kernel(
        _sc_msg_body,
        out_type=[
            jax.ShapeDtypeStruct((NC, N, D), f32),
            jax.ShapeDtypeStruct((NC, N, D), f32),
            jax.ShapeDtypeStruct((NC, N, L), f32),
        ],
        mesh=mesh,
        scratch_types=[
            pltpu.VMEM((2, _C2), i32),
            pltpu.VMEM((2, _C2), i32),
            pltpu.VMEM((_C2,), f32),
            pltpu.VMEM((_C2,), f32),
            pltpu.VMEM((_C2, 2 * D), f32),
            pltpu.VMEM((_C2, 2 * D), f32),
            pltpu.VMEM((_C2, 2 * D), f32),
            pltpu.VMEM((_C2, 2 * D), f32),
            pltpu.VMEM((_C2, D), f32),
            pltpu.VMEM((_C2, D), f32),
            pltpu.VMEM((_C2, D), f32),
            pltpu.VMEM((_C2, D), f32),
            pltpu.VMEM((_C2, L), f32),
            pltpu.VMEM_SHARED((N, D), f32),
            pltpu.VMEM_SHARED((N, D), f32),
            pltpu.VMEM_SHARED((N, L), f32),
            pltpu.SemaphoreType.DMA,
            pltpu.SemaphoreType.DMA,
            pltpu.SemaphoreType.DMA,
            pltpu.SemaphoreType.DMA,
            pltpu.SemaphoreType.DMA,
            pltpu.SemaphoreType.DMA,
            pltpu.SemaphoreType.DMA,
            pltpu.SemaphoreType.DMA,
            pltpu.SemaphoreType.DMA,
            pltpu.SemaphoreType.DMA,
        ],
    )
    return f(pdx, psx, idxpk, wpk, zrow, zcnt)


def _t3_body(accx, acce, cnt, x_blk, emb_blk, wn2, wa2, wt1, wt2, wm1, wm2,
             oval, oatt):
    cx = accx[0] + accx[1]
    ce = acce[0] + acce[1]
    cdeg = cnt[0, :, 0:1] + cnt[1, :, 0:1]
    meanx = _dot(cx, wn2[...]) / cdeg
    h = _dot(meanx, wt1[0:D, :]) + _dot(x_blk[...], wt1[D:2 * D, :])
    h = jnp.where(h >= 0, h, NEG * h)
    oval[...] = _dot(h, wt2[...])
    meane = _dot(ce, wa2[...]) / cdeg
    g = _dot(meane, wm1[0:D, :]) + _dot(emb_blk[...], wm1[D:2 * D, :])
    g = jnp.where(g >= 0, g, NEG * g)
    oatt[...] = _dot(g, wm2[...])


def _t3(accx, acce, cnt, x, emb, wn2, wa2, wt1, wt2, wm1, wm2):
    f32 = jnp.float32
    nb = N // BR3
    return pl.pallas_call(
        _t3_body,
        grid=(nb,),
        in_specs=[
            pl.BlockSpec((NC, BR3, D), lambda i: (0, i, 0)),
            pl.BlockSpec((NC, BR3, D), lambda i: (0, i, 0)),
            pl.BlockSpec((NC, BR3, L), lambda i: (0, i, 0)),
            pl.BlockSpec((BR3, D), lambda i: (i, 0)),
            pl.BlockSpec((BR3, D), lambda i: (i, 0)),
            pl.BlockSpec((D, D), lambda i: (0, 0)),
            pl.BlockSpec((D, D), lambda i: (0, 0)),
            pl.BlockSpec((2 * D, D), lambda i: (0, 0)),
            pl.BlockSpec((D, OUT), lambda i: (0, 0)),
            pl.BlockSpec((2 * D, D), lambda i: (0, 0)),
            pl.BlockSpec((D, OUT), lambda i: (0, 0)),
        ],
        out_specs=[
            pl.BlockSpec((BR3, OUT), lambda i: (i, 0)),
            pl.BlockSpec((BR3, OUT), lambda i: (i, 0)),
        ],
        out_shape=[
            jax.ShapeDtypeStruct((N, OUT), f32),
            jax.ShapeDtypeStruct((N, OUT), f32),
        ],
    )(accx, acce, cnt, x, emb, wn2, wa2, wt1, wt2, wm1, wm2)


def kernel(x, decoupled_emb, edge_index, mask, attention_init,
           Wq, Wk, Wn1, Wn2, Wa1, Wa2, Wt1, Wt2, Wm1, Wm2):
    emb = decoupled_emb
    (scores, wnew, topi, diag, ent,
     pdx, psx) = _t1(emb, x, Wq, Wk, Wn1, Wa1)

    src = edge_index[0]
    dst = edge_index[1]
    w_orig = _sc_w(scores, src, dst)

    loop = jnp.arange(N, dtype=jnp.int32)
    all_src = jnp.concatenate([src, jnp.repeat(loop, K), loop])
    all_dst = jnp.concatenate([dst, topi.reshape(-1), loop])
    all_w = jnp.concatenate([w_orig, wnew.reshape(-1), diag.reshape(-1)])

    accx, acce, cnt = _sc_msg(pdx, psx, all_src, all_dst, all_w)
    out_val, out_att = _t3(accx, acce, cnt, x, emb,
                           Wn2, Wa2, Wt1, Wt2, Wm1, Wm2)

    updated_edge_index = jnp.stack([all_src, all_dst])
    edge_penalty = ent[0, 0]
    return out_val, updated_edge_index, edge_penalty, scores, out_att, all_w
